# baseline (speedup 1.0000x reference)
import functools

import jax
import jax.numpy as jnp
from jax import lax
from jax.experimental import pallas as pl
from jax.experimental.pallas import tpu as pltpu
from jax.experimental.pallas import tpu_sc as plsc

B = 16384
CTX = 8
NSAMP = 10
D = 64
NC = 2
NS = 16
NW = NC * NS
BW = B // NW
SUP = 128
CHUNK = 32
NSUP = BW // SUP
KPS = SUP // CHUNK
NG = CHUNK // 16


def _body(doc_hbm, ctx_hbm, smp_hbm, par_hbm, word_hbm, out_hbm, res_hbm,
          doc_raw, ctx_raw, smp_raw, doc_sh, ctx_sh, smp_sh,
          doc_rows, ctx_rows, smp_rows, inp_v, res_v, sem):
    wid = lax.axis_index("s") * NC + lax.axis_index("c")
    lanes = lax.iota(jnp.int32, 16)
    lanes10 = lanes * NSAMP

    def sup_body(j, carry):
        base = pl.multiple_of(wid * BW + j * SUP, SUP)
        pltpu.sync_copy(doc_hbm.at[pl.ds(base, SUP)], doc_raw)
        pltpu.sync_copy(ctx_hbm.at[:, pl.ds(base, SUP)], ctx_raw)
        pltpu.sync_copy(smp_hbm.at[:, pl.ds(base, SUP)], smp_raw)

        for k in range(KPS):
            for t in range(CHUNK // 16):
                doc_sh[pl.ds(t * 16, 16)] = (
                    doc_raw[pl.ds(k * CHUNK + t * 16, 16)] >> 1)
                for c in range(CTX):
                    ctx_sh[pl.ds(c * CHUNK + t * 16, 16)] = (
                        ctx_raw[c, pl.ds(k * CHUNK + t * 16, 16)] >> 1)
                for s in range(NSAMP):
                    smp_sh[pl.ds(s * CHUNK + t * 16, 16)] = (
                        smp_raw[s, pl.ds(k * CHUNK + t * 16, 16)] >> 1)
            copies = [pltpu.async_copy(par_hbm.at[doc_sh], doc_rows, sem)]
            for c in range(CTX):
                copies.append(pltpu.async_copy(
                    word_hbm.at[ctx_sh.at[pl.ds(c * CHUNK, CHUNK)]],
                    ctx_rows.at[pl.ds(c * CHUNK, CHUNK)], sem))
            for s in range(NSAMP):
                copies.append(pltpu.async_copy(
                    out_hbm.at[smp_sh.at[pl.ds(s * CHUNK, CHUNK)]],
                    smp_rows.at[pl.ds(s * CHUNK, CHUNK)], sem))
            for cp in copies:
                cp.wait()

            for g in range(NG):
                egl = k * CHUNK + g * 16
                drow = g * 16 + lanes
                dcol0 = (doc_raw[pl.ds(egl, 16)] & 1) << 6
                ccols0 = [(ctx_raw[c, pl.ds(egl, 16)] & 1) << 6
                          for c in range(CTX)]
                crows = [c * CHUNK + g * 16 + lanes for c in range(CTX)]

                def p1(d, cols):
                    dcol, ccols = cols[0], cols[1:]
                    a = plsc.load_gather(ctx_rows, [crows[0], ccols[0]])
                    for c in range(1, CTX):
                        a = a + plsc.load_gather(ctx_rows,
                                                 [crows[c], ccols[c]])
                    v = plsc.load_gather(doc_rows, [drow, dcol])
                    v = v + a * (1.0 / CTX)
                    inp_v[pl.ds(d * 16, 16)] = v
                    return tuple(x + 1 for x in cols)

                lax.fori_loop(0, D, p1, (dcol0,) + tuple(ccols0))

                scols0 = [(smp_raw[s, pl.ds(egl, 16)] & 1) << 6
                          for s in range(NSAMP)]
                srows = [s * CHUNK + g * 16 + lanes for s in range(NSAMP)]
                zeros = jnp.zeros((16,), jnp.float32)

                def p2(d, carry2):
                    accs, scols = carry2
                    v = inp_v[pl.ds(d * 16, 16)]
                    new_accs = tuple(
                        accs[s] + v * plsc.load_gather(smp_rows,
                                                       [srows[s], scols[s]])
                        for s in range(NSAMP))
                    return new_accs, tuple(x + 1 for x in scols)

                accs, _ = lax.fori_loop(
                    0, D, p2, ((zeros,) * NSAMP, tuple(scols0)))

                rbase = (j * SUP + egl) * NSAMP
                for s in range(NSAMP):
                    plsc.store_scatter(res_v, [lanes10 + (rbase + s)],
                                       accs[s])
        return carry

    lax.fori_loop(0, NSUP, sup_body, 0)
    pltpu.sync_copy(res_v, res_hbm.at[pl.ds(wid * (BW * NSAMP), BW * NSAMP)])


_sc_call = functools.partial(
    pl.kernel,
    out_type=jax.ShapeDtypeStruct((B * NSAMP,), jnp.float32),
    mesh=plsc.VectorSubcoreMesh(core_axis_name="c", subcore_axis_name="s"),
    compiler_params=pltpu.CompilerParams(needs_layout_passes=False,
                                         use_tc_tiling_on_sc=True),
    scratch_types=[
        pltpu.VMEM((SUP,), jnp.int32),
        pltpu.VMEM((CTX, SUP), jnp.int32),
        pltpu.VMEM((NSAMP, SUP), jnp.int32),
        pltpu.VMEM((CHUNK,), jnp.int32),
        pltpu.VMEM((CTX * CHUNK,), jnp.int32),
        pltpu.VMEM((NSAMP * CHUNK,), jnp.int32),
        pltpu.VMEM((CHUNK, 128), jnp.float32),
        pltpu.VMEM((CTX * CHUNK, 128), jnp.float32),
        pltpu.VMEM((NSAMP * CHUNK, 128), jnp.float32),
        pltpu.VMEM((D * 16,), jnp.float32),
        pltpu.VMEM((BW * NSAMP,), jnp.float32),
        pltpu.SemaphoreType.DMA,
    ],
)(_body)


def kernel(doc_ids, context_ids, sample_ids, paragraph_matrix, word_matrix,
           outputs):
    doc32 = doc_ids.astype(jnp.int32)
    par2 = paragraph_matrix.reshape(-1, 128)
    word2 = word_matrix.reshape(-1, 128)
    out2 = outputs.T.reshape(-1, 128)
    res = _sc_call(doc32, context_ids.T, sample_ids.T, par2, word2, out2)
    return res.reshape(B, NSAMP)


# speedup vs baseline: 1.0118x; 1.0118x over previous
import functools

import jax
import jax.numpy as jnp
from jax import lax
from jax.experimental import pallas as pl
from jax.experimental.pallas import tpu as pltpu
from jax.experimental.pallas import tpu_sc as plsc

B = 16384
CTX = 8
NSAMP = 10
D = 64
NC = 2
NS = 16
NW = NC * NS
BW = B // NW
SUP = 128
CHUNK = 32
NSUP = BW // SUP
KPS = SUP // CHUNK
NG = CHUNK // 16


def _body(doc_hbm, ctx_hbm, smp_hbm, par_hbm, word_hbm, out_hbm, res_hbm,
          doc_raw, ctx_raw, smp_raw, doc_sh, ctx_sh, smp_sh,
          doc_rows, ctx_rows, smp_rows, inp_v, res_v, sem):
    wid = lax.axis_index("s") * NC + lax.axis_index("c")
    lanes = lax.iota(jnp.int32, 16)
    lanes10 = lanes * NSAMP

    def sup_body(j, carry):
        base = pl.multiple_of(wid * BW + j * SUP, SUP)
        pltpu.sync_copy(doc_hbm.at[pl.ds(base, SUP)], doc_raw)
        pltpu.sync_copy(ctx_hbm.at[:, pl.ds(base, SUP)], ctx_raw)
        pltpu.sync_copy(smp_hbm.at[:, pl.ds(base, SUP)], smp_raw)

        for k in range(KPS):
            for t in range(CHUNK // 16):
                doc_sh[pl.ds(t * 16, 16)] = (
                    doc_raw[pl.ds(k * CHUNK + t * 16, 16)] >> 1)
                for c in range(CTX):
                    ctx_sh[pl.ds(c * CHUNK + t * 16, 16)] = (
                        ctx_raw[c, pl.ds(k * CHUNK + t * 16, 16)] >> 1)
                for s in range(NSAMP):
                    smp_sh[pl.ds(s * CHUNK + t * 16, 16)] = (
                        smp_raw[s, pl.ds(k * CHUNK + t * 16, 16)] >> 1)
            copies = [pltpu.async_copy(par_hbm.at[doc_sh], doc_rows, sem)]
            for c in range(CTX):
                copies.append(pltpu.async_copy(
                    word_hbm.at[ctx_sh.at[pl.ds(c * CHUNK, CHUNK)]],
                    ctx_rows.at[pl.ds(c * CHUNK, CHUNK)], sem))
            for s in range(NSAMP):
                copies.append(pltpu.async_copy(
                    out_hbm.at[smp_sh.at[pl.ds(s * CHUNK, CHUNK)]],
                    smp_rows.at[pl.ds(s * CHUNK, CHUNK)], sem))
            for cp in copies:
                cp.wait()

            for g in range(NG):
                egl = k * CHUNK + g * 16
                drow = g * 16 + lanes
                dcol0 = (doc_raw[pl.ds(egl, 16)] & 1) << 6
                ccols0 = [(ctx_raw[c, pl.ds(egl, 16)] & 1) << 6
                          for c in range(CTX)]
                crows = [c * CHUNK + g * 16 + lanes for c in range(CTX)]

                UNR = 8

                def p1u(i, carry2):
                    d0 = i * UNR
                    for u in range(UNR):
                        a = plsc.load_gather(
                            ctx_rows, [crows[0], ccols0[0] + (d0 + u)])
                        for c in range(1, CTX):
                            a = a + plsc.load_gather(
                                ctx_rows, [crows[c], ccols0[c] + (d0 + u)])
                        v = plsc.load_gather(doc_rows,
                                             [drow, dcol0 + (d0 + u)])
                        inp_v[pl.ds((d0 + u) * 16, 16)] = (
                            v + a * (1.0 / CTX))
                    return carry2

                lax.fori_loop(0, D // UNR, p1u, 0)

                scols0 = [(smp_raw[s, pl.ds(egl, 16)] & 1) << 6
                          for s in range(NSAMP)]
                srows = [s * CHUNK + g * 16 + lanes for s in range(NSAMP)]
                zeros = jnp.zeros((16,), jnp.float32)
                rbase = (j * SUP + egl) * NSAMP
                for h in range(2):
                    sl = range(h * 5, h * 5 + 5)

                    def p2(i, accs, sl=sl):
                        d0 = i * UNR
                        accs = list(accs)
                        for u in range(UNR):
                            v = inp_v[pl.ds((d0 + u) * 16, 16)]
                            for o, s in enumerate(sl):
                                accs[o] = accs[o] + v * plsc.load_gather(
                                    smp_rows, [srows[s],
                                               scols0[s] + (d0 + u)])
                        return tuple(accs)

                    accs = lax.fori_loop(0, D // UNR, p2, (zeros,) * 5)
                    for o, s in enumerate(sl):
                        plsc.store_scatter(res_v, [lanes10 + (rbase + s)],
                                           accs[o])
        return carry

    lax.fori_loop(0, NSUP, sup_body, 0)
    pltpu.sync_copy(res_v, res_hbm.at[pl.ds(wid * (BW * NSAMP), BW * NSAMP)])


_sc_call = functools.partial(
    pl.kernel,
    out_type=jax.ShapeDtypeStruct((B * NSAMP,), jnp.float32),
    mesh=plsc.VectorSubcoreMesh(core_axis_name="c", subcore_axis_name="s"),
    compiler_params=pltpu.CompilerParams(needs_layout_passes=False,
                                         use_tc_tiling_on_sc=True),
    scratch_types=[
        pltpu.VMEM((SUP,), jnp.int32),
        pltpu.VMEM((CTX, SUP), jnp.int32),
        pltpu.VMEM((NSAMP, SUP), jnp.int32),
        pltpu.VMEM((CHUNK,), jnp.int32),
        pltpu.VMEM((CTX * CHUNK,), jnp.int32),
        pltpu.VMEM((NSAMP * CHUNK,), jnp.int32),
        pltpu.VMEM((CHUNK, 128), jnp.float32),
        pltpu.VMEM((CTX * CHUNK, 128), jnp.float32),
        pltpu.VMEM((NSAMP * CHUNK, 128), jnp.float32),
        pltpu.VMEM((D * 16,), jnp.float32),
        pltpu.VMEM((BW * NSAMP,), jnp.float32),
        pltpu.SemaphoreType.DMA,
    ],
)(_body)


def kernel(doc_ids, context_ids, sample_ids, paragraph_matrix, word_matrix,
           outputs):
    doc32 = doc_ids.astype(jnp.int32)
    par2 = paragraph_matrix.reshape(-1, 128)
    word2 = word_matrix.reshape(-1, 128)
    out2 = outputs.T.reshape(-1, 128)
    res = _sc_call(doc32, context_ids.T, sample_ids.T, par2, word2, out2)
    return res.reshape(B, NSAMP)


# speedup vs baseline: 1.2783x; 1.2634x over previous
import functools

import jax
import jax.numpy as jnp
from jax import lax
from jax.experimental import pallas as pl
from jax.experimental.pallas import tpu as pltpu
from jax.experimental.pallas import tpu_sc as plsc

B = 16384
CTX = 8
NSAMP = 10
D = 64
NDOC = 1000000
NWORD = 100000
NC = 2
NS = 16
NW = NC * NS
BW = B // NW

WREP = NWORD + 96
DSTG = B + NW
W_NWIN = 99
D_NWIN = 978
BNDPAD = 1088

_i32 = jnp.int32


def _splat(x):
    return jnp.full((16,), x, _i32)


def _a_body(parT, wordT, outsN, auxp, auxw, auxo, jsort, bsort, bnd,
            doc_stage, word_rep, out_rep,
            slab, asm, asm_doc, sidx, jsv, bsv, bndv, sem):
    wid = lax.axis_index("s") * NC + lax.axis_index("c")
    lanes = lax.iota(_i32, 16)

    def transpose_job(src, aux, dst):
        def win_body(m, carry):
            gw = wid + m * 32

            @pl.when(gw < W_NWIN)
            def _():
                tail = gw == W_NWIN - 1
                w = jnp.where(tail, 99968,
                              jnp.minimum(gw * 1024, 98944))
                w = pl.multiple_of(w, 128)

                @pl.when(jnp.logical_not(tail))
                def _():
                    pltpu.sync_copy(src.at[:, pl.ds(w, 1024)],
                                    slab.at[:, pl.ds(0, 1024)])

                @pl.when(tail)
                def _():
                    pltpu.sync_copy(aux, slab.at[:, pl.ds(0, 128)])

                nblk = jnp.where(tail, 1, 4)

                def blk_body(blk, c2):
                    njb = jnp.where(tail, 4, 16)

                    def jb_body(jb, c3):
                        jl0 = blk * 256 + jb * 16
                        for r in range(16):
                            for t in range(4):
                                g = plsc.load_gather(
                                    slab, [t * 16 + lanes, _splat(jl0 + r)])
                                asm[jb * 16 + r, pl.ds(t * 16, 16)] = g
                                asm[jb * 16 + r, pl.ds(64 + t * 16, 16)] = g
                        return c3

                    lax.fori_loop(0, njb, jb_body, 0)

                    @pl.when(tail)
                    def _():
                        pltpu.sync_copy(asm.at[pl.ds(0, 64)],
                                        dst.at[pl.ds(w, 64)])

                    @pl.when(jnp.logical_not(tail))
                    def _():
                        pltpu.sync_copy(asm, dst.at[pl.ds(w + blk * 256,
                                                          256)])
                    return c2

                lax.fori_loop(0, nblk, blk_body, 0)
            return carry

        lax.fori_loop(0, 4, win_body, 0)

    transpose_job(wordT, auxw, word_rep)
    transpose_job(outsN, auxo, out_rep)

    pltpu.sync_copy(bnd.at[pl.ds(wid * 32, 64)], bndv)
    bvecs = [bndv[pl.ds(q * 16, 16)] for q in range(3)]
    lo_t = jnp.sum(jnp.where(lanes == 0, bvecs[0], 0))
    stage0 = jnp.clip(lo_t & ~7, 0, B - 2048)
    stage0 = pl.multiple_of(stage0, 8)
    pltpu.sync_copy(jsort.at[pl.ds(stage0, 2048)], jsv)
    pltpu.sync_copy(bsort.at[pl.ds(stage0, 2048)], bsv)

    def ext(i):
        r = jnp.zeros((), _i32)
        for q in range(3):
            r = r + jnp.sum(jnp.where(lanes + 16 * q == i, bvecs[q], 0))
        return r

    def dwin_body(m, carry):
        gw = wid * 32 + m

        @pl.when(gw < D_NWIN)
        def _():
            lo = ext(m)
            hi = ext(m + 1)
            w = jnp.where(gw <= 975, gw * 1024,
                          jnp.where(gw == 976, 998912, 999936))
            w = pl.multiple_of(w, 128)

            @pl.when(gw <= 976)
            def _():
                pltpu.sync_copy(parT.at[:, pl.ds(w, 1024)],
                                slab.at[:, pl.ds(0, 1024)])

            @pl.when(gw == 977)
            def _():
                pltpu.sync_copy(auxp, slab.at[:, pl.ds(0, 128)])

            nb = (hi - lo + 15) // 16

            def batch_body(q, c2):
                pos = lo + q * 16
                off = pos - stage0
                jv = jsv[pl.ds(off, 16)]
                bv = bsv[pl.ds(off, 16)]
                msk = (pos + lanes) < hi
                jloc = jnp.clip(jv - w, 0, 1023)
                for d in range(D):
                    g = plsc.load_gather(slab, [_splat(d), jloc])
                    plsc.store_scatter(asm_doc, [lanes, _splat(d)], g)
                sidx[...] = jnp.where(msk, bv, B + wid)
                pltpu.sync_copy(asm_doc.at[:, pl.ds(0, 128)],
                                doc_stage.at[sidx])
                return c2

            lax.fori_loop(0, nb, batch_body, 0)
        return carry

    lax.fori_loop(0, 32, dwin_body, 0)


_a_call = functools.partial(
    pl.kernel,
    out_type=(jax.ShapeDtypeStruct((DSTG, 128), jnp.float32),
              jax.ShapeDtypeStruct((WREP, 128), jnp.float32),
              jax.ShapeDtypeStruct((WREP, 128), jnp.float32)),
    mesh=plsc.VectorSubcoreMesh(core_axis_name="c", subcore_axis_name="s"),
    compiler_params=pltpu.CompilerParams(needs_layout_passes=False,
                                         use_tc_tiling_on_sc=True),
    scratch_types=[
        pltpu.VMEM((D, 1025), jnp.float32),
        pltpu.VMEM((256, 128), jnp.float32),
        pltpu.VMEM((16, 129), jnp.float32),
        pltpu.VMEM((16,), _i32),
        pltpu.VMEM((2048,), _i32),
        pltpu.VMEM((2048,), _i32),
        pltpu.VMEM((64,), _i32),
        pltpu.SemaphoreType.DMA,
    ],
)(_a_body)


SUP = 128
CHUNK = 32
NSUPB = BW // SUP
KPS = SUP // CHUNK


def _b_body(ctx_hbm, smp_hbm, doc_stage, word_rep, out_rep, res_hbm,
            ctx_raw, smp_raw, ctx_idx, smp_idx,
            doc_rows, ctx_rows, smp_rows, res_v, sem):
    wid = lax.axis_index("s") * NC + lax.axis_index("c")
    lanes = lax.iota(_i32, 16)

    def sup_body(j, carry):
        base = pl.multiple_of(wid * BW + j * SUP, SUP)
        pltpu.sync_copy(ctx_hbm.at[:, pl.ds(base, SUP)], ctx_raw)
        pltpu.sync_copy(smp_hbm.at[:, pl.ds(base, SUP)], smp_raw)

        for k in range(KPS):
            for t in range(CHUNK // 16):
                for c in range(CTX):
                    ctx_idx[pl.ds(c * CHUNK + t * 16, 16)] = (
                        ctx_raw[c, pl.ds(k * CHUNK + t * 16, 16)])
                for s in range(NSAMP):
                    smp_idx[pl.ds(s * CHUNK + t * 16, 16)] = (
                        smp_raw[s, pl.ds(k * CHUNK + t * 16, 16)])
            copies = [pltpu.async_copy(
                doc_stage.at[pl.ds(base + k * CHUNK, CHUNK)], doc_rows, sem)]
            for c in range(CTX):
                copies.append(pltpu.async_copy(
                    word_rep.at[ctx_idx.at[pl.ds(c * CHUNK, CHUNK)]],
                    ctx_rows.at[pl.ds(c * CHUNK, CHUNK)], sem))
            for s in range(NSAMP):
                copies.append(pltpu.async_copy(
                    out_rep.at[smp_idx.at[pl.ds(s * CHUNK, CHUNK)]],
                    smp_rows.at[pl.ds(s * CHUNK, CHUNK)], sem))
            for cp in copies:
                cp.wait()

            rb0 = (j * SUP + k * CHUNK) * NSAMP

            def elem_body(e, c2):
                inp = []
                for t in range(4):
                    a = ctx_rows[e, pl.ds(t * 16, 16)]
                    for c in range(1, CTX):
                        a = a + ctx_rows[c * CHUNK + e, pl.ds(t * 16, 16)]
                    inp.append(doc_rows[e, pl.ds(t * 16, 16)]
                               + a * (1.0 / CTX))
                acc = jnp.zeros((16,), jnp.float32)
                for s in range(NSAMP):
                    r = inp[0] * smp_rows[s * CHUNK + e, pl.ds(0, 16)]
                    for t in range(1, 4):
                        r = r + inp[t] * smp_rows[s * CHUNK + e,
                                                  pl.ds(t * 16, 16)]
                    acc = jnp.where(lanes == s, jnp.sum(r), acc)
                plsc.store_scatter(res_v, [rb0 + e * NSAMP + lanes], acc,
                                   mask=lanes < NSAMP)
                return c2

            lax.fori_loop(0, CHUNK, elem_body, 0)
        return carry

    lax.fori_loop(0, NSUPB, sup_body, 0)
    pltpu.sync_copy(res_v,
                    res_hbm.at[pl.ds(wid * (BW * NSAMP), BW * NSAMP)])


_b_call = functools.partial(
    pl.kernel,
    out_type=jax.ShapeDtypeStruct((B * NSAMP,), jnp.float32),
    mesh=plsc.VectorSubcoreMesh(core_axis_name="c", subcore_axis_name="s"),
    compiler_params=pltpu.CompilerParams(needs_layout_passes=False,
                                         use_tc_tiling_on_sc=True),
    scratch_types=[
        pltpu.VMEM((CTX, SUP), _i32),
        pltpu.VMEM((NSAMP, SUP), _i32),
        pltpu.VMEM((CTX * CHUNK,), _i32),
        pltpu.VMEM((NSAMP * CHUNK,), _i32),
        pltpu.VMEM((CHUNK, 128), jnp.float32),
        pltpu.VMEM((CTX * CHUNK, 128), jnp.float32),
        pltpu.VMEM((NSAMP * CHUNK, 128), jnp.float32),
        pltpu.VMEM((BW * NSAMP,), jnp.float32),
        pltpu.SemaphoreType.DMA,
    ],
)(_b_body)


def kernel(doc_ids, context_ids, sample_ids, paragraph_matrix, word_matrix,
           outputs):
    f32 = jnp.float32
    doc32 = doc_ids.astype(_i32)
    jsorted, order = lax.sort_key_val(doc32, jnp.arange(B, dtype=_i32))
    edges = jnp.concatenate([
        jnp.arange(0, 977 * 1024, 1024, dtype=_i32),
        jnp.array([999936, NDOC], dtype=_i32)])
    bnd = jnp.searchsorted(jsorted, edges).astype(_i32)
    bnd = jnp.pad(bnd, (0, BNDPAD - bnd.shape[0]), constant_values=B)
    auxp = jnp.concatenate(
        [paragraph_matrix[999936:].T, jnp.zeros((D, 64), f32)], axis=1)
    auxw = jnp.concatenate(
        [word_matrix[99968:].T, jnp.zeros((D, 96), f32)], axis=1)
    auxo = jnp.concatenate(
        [outputs[:, 99968:], jnp.zeros((D, 96), f32)], axis=1)
    doc_stage, word_rep, out_rep = _a_call(
        paragraph_matrix.T, word_matrix.T, outputs, auxp, auxw, auxo,
        jsorted, order, bnd)
    res = _b_call(context_ids.T, sample_ids.T, doc_stage, word_rep, out_rep)
    return res.reshape(B, NSAMP)


# speedup vs baseline: 1.2814x; 1.0025x over previous
import functools

import jax
import jax.numpy as jnp
from jax import lax
from jax.experimental import pallas as pl
from jax.experimental.pallas import tpu as pltpu
from jax.experimental.pallas import tpu_sc as plsc

B = 16384
CTX = 8
NSAMP = 10
D = 64
NDOC = 1000000
NWORD = 100000
NC = 2
NS = 16
NW = NC * NS
BW = B // NW

WREP = NWORD + 96
DSTG = B + NW
W_NWIN = 99
D_NWIN = 978
BNDPAD = 1088

_i32 = jnp.int32


def _splat(x):
    return jnp.full((16,), x, _i32)


def _a_body(parT, wordT, outsN, auxp, auxw, auxo, jsort, bsort, bnd,
            doc_stage, word_rep, out_rep,
            slab, asm, asm_doc, sidx, jsv, bsv, bndv, sem):
    wid = lax.axis_index("s") * NC + lax.axis_index("c")
    lanes = lax.iota(_i32, 16)

    def transpose_job(src, aux, dst):
        def win_body(m, carry):
            gw = wid + m * 32

            @pl.when(gw < W_NWIN)
            def _():
                tail = gw == W_NWIN - 1
                w = jnp.where(tail, 99968,
                              jnp.minimum(gw * 1024, 98944))
                w = pl.multiple_of(w, 128)

                @pl.when(jnp.logical_not(tail))
                def _():
                    pltpu.sync_copy(src.at[:, pl.ds(w, 1024)],
                                    slab.at[:, pl.ds(0, 1024)])

                @pl.when(tail)
                def _():
                    pltpu.sync_copy(aux, slab.at[:, pl.ds(0, 128)])

                nblk = jnp.where(tail, 1, 4)

                def blk_body(blk, c2):
                    njb = jnp.where(tail, 4, 16)

                    def jb_body(jb, c3):
                        jl0 = blk * 256 + jb * 16
                        for r in range(16):
                            for t in range(4):
                                g = plsc.load_gather(
                                    slab, [t * 16 + lanes, _splat(jl0 + r)])
                                asm[jb * 16 + r, pl.ds(t * 16, 16)] = g
                                asm[jb * 16 + r, pl.ds(64 + t * 16, 16)] = g
                        return c3

                    lax.fori_loop(0, njb, jb_body, 0)

                    @pl.when(tail)
                    def _():
                        pltpu.sync_copy(asm.at[pl.ds(0, 64)],
                                        dst.at[pl.ds(w, 64)])

                    @pl.when(jnp.logical_not(tail))
                    def _():
                        pltpu.sync_copy(asm, dst.at[pl.ds(w + blk * 256,
                                                          256)])
                    return c2

                lax.fori_loop(0, nblk, blk_body, 0)
            return carry

        lax.fori_loop(0, 4, win_body, 0)

    transpose_job(wordT, auxw, word_rep)
    transpose_job(outsN, auxo, out_rep)

    pltpu.sync_copy(bnd.at[pl.ds(wid * 32, 64)], bndv)
    bvecs = [bndv[pl.ds(q * 16, 16)] for q in range(3)]
    lo_t = jnp.sum(jnp.where(lanes == 0, bvecs[0], 0))
    stage0 = jnp.clip(lo_t & ~7, 0, B - 2048)
    stage0 = pl.multiple_of(stage0, 8)
    pltpu.sync_copy(jsort.at[pl.ds(stage0, 2048)], jsv)
    pltpu.sync_copy(bsort.at[pl.ds(stage0, 2048)], bsv)

    def ext(i):
        r = jnp.zeros((), _i32)
        for q in range(3):
            r = r + jnp.sum(jnp.where(lanes + 16 * q == i, bvecs[q], 0))
        return r

    def dwin_body(m, carry):
        gw = wid * 32 + m

        @pl.when(gw < D_NWIN)
        def _():
            lo = ext(m)
            hi = ext(m + 1)
            w = jnp.where(gw <= 975, gw * 1024,
                          jnp.where(gw == 976, 998912, 999936))
            w = pl.multiple_of(w, 128)

            @pl.when(gw <= 976)
            def _():
                pltpu.sync_copy(parT.at[:, pl.ds(w, 1024)],
                                slab.at[:, pl.ds(0, 1024)])

            @pl.when(gw == 977)
            def _():
                pltpu.sync_copy(auxp, slab.at[:, pl.ds(0, 128)])

            nb = (hi - lo + 15) // 16

            def batch_body(q, c2):
                pos = lo + q * 16
                off = pos - stage0
                jv = jsv[pl.ds(off, 16)]
                bv = bsv[pl.ds(off, 16)]
                msk = (pos + lanes) < hi
                jloc = jnp.clip(jv - w, 0, 1023)
                for d in range(D):
                    g = plsc.load_gather(slab, [_splat(d), jloc])
                    plsc.store_scatter(asm_doc, [lanes, _splat(d)], g)
                sidx[...] = jnp.where(msk, bv, B + wid)
                pltpu.sync_copy(asm_doc.at[:, pl.ds(0, 128)],
                                doc_stage.at[sidx])
                return c2

            lax.fori_loop(0, nb, batch_body, 0)
        return carry

    lax.fori_loop(0, 32, dwin_body, 0)


_a_call = functools.partial(
    pl.kernel,
    out_type=(jax.ShapeDtypeStruct((DSTG, 128), jnp.float32),
              jax.ShapeDtypeStruct((WREP, 128), jnp.float32),
              jax.ShapeDtypeStruct((WREP, 128), jnp.float32)),
    mesh=plsc.VectorSubcoreMesh(core_axis_name="c", subcore_axis_name="s"),
    compiler_params=pltpu.CompilerParams(needs_layout_passes=False,
                                         use_tc_tiling_on_sc=True),
    scratch_types=[
        pltpu.VMEM((D, 1032), jnp.float32),
        pltpu.VMEM((256, 128), jnp.float32),
        pltpu.VMEM((16, 136), jnp.float32),
        pltpu.VMEM((16,), _i32),
        pltpu.VMEM((2048,), _i32),
        pltpu.VMEM((2048,), _i32),
        pltpu.VMEM((64,), _i32),
        pltpu.SemaphoreType.DMA,
    ],
)(_a_body)


SUP = 128
CHUNK = 32
NSUPB = BW // SUP
KPS = SUP // CHUNK


def _b_body(ctx_hbm, smp_hbm, doc_stage, word_rep, out_rep, res_hbm,
            ctx_raw, smp_raw, ctx_idx, smp_idx,
            doc_rows, ctx_rows, smp_rows, res_v, sem):
    wid = lax.axis_index("s") * NC + lax.axis_index("c")
    lanes = lax.iota(_i32, 16)

    def sup_body(j, carry):
        base = pl.multiple_of(wid * BW + j * SUP, SUP)
        pltpu.sync_copy(ctx_hbm.at[:, pl.ds(base, SUP)], ctx_raw)
        pltpu.sync_copy(smp_hbm.at[:, pl.ds(base, SUP)], smp_raw)

        for k in range(KPS):
            for t in range(CHUNK // 16):
                for c in range(CTX):
                    ctx_idx[pl.ds(c * CHUNK + t * 16, 16)] = (
                        ctx_raw[c, pl.ds(k * CHUNK + t * 16, 16)])
                for s in range(NSAMP):
                    smp_idx[pl.ds(s * CHUNK + t * 16, 16)] = (
                        smp_raw[s, pl.ds(k * CHUNK + t * 16, 16)])
            copies = [pltpu.async_copy(
                doc_stage.at[pl.ds(base + k * CHUNK, CHUNK)], doc_rows, sem)]
            for c in range(CTX):
                copies.append(pltpu.async_copy(
                    word_rep.at[ctx_idx.at[pl.ds(c * CHUNK, CHUNK)]],
                    ctx_rows.at[pl.ds(c * CHUNK, CHUNK)], sem))
            for s in range(NSAMP):
                copies.append(pltpu.async_copy(
                    out_rep.at[smp_idx.at[pl.ds(s * CHUNK, CHUNK)]],
                    smp_rows.at[pl.ds(s * CHUNK, CHUNK)], sem))
            for cp in copies:
                cp.wait()

            rb0 = (j * SUP + k * CHUNK) * NSAMP

            def elem_body(e, c2):
                inp = []
                for t in range(4):
                    a = ctx_rows[e, pl.ds(t * 16, 16)]
                    for c in range(1, CTX):
                        a = a + ctx_rows[c * CHUNK + e, pl.ds(t * 16, 16)]
                    inp.append(doc_rows[e, pl.ds(t * 16, 16)]
                               + a * (1.0 / CTX))
                acc = jnp.zeros((16,), jnp.float32)
                for s in range(NSAMP):
                    r = inp[0] * smp_rows[s * CHUNK + e, pl.ds(0, 16)]
                    for t in range(1, 4):
                        r = r + inp[t] * smp_rows[s * CHUNK + e,
                                                  pl.ds(t * 16, 16)]
                    acc = jnp.where(lanes == s, jnp.sum(r), acc)
                plsc.store_scatter(res_v, [rb0 + e * NSAMP + lanes], acc,
                                   mask=lanes < NSAMP)
                return c2

            lax.fori_loop(0, CHUNK, elem_body, 0)
        return carry

    lax.fori_loop(0, NSUPB, sup_body, 0)
    pltpu.sync_copy(res_v,
                    res_hbm.at[pl.ds(wid * (BW * NSAMP), BW * NSAMP)])


_b_call = functools.partial(
    pl.kernel,
    out_type=jax.ShapeDtypeStruct((B * NSAMP,), jnp.float32),
    mesh=plsc.VectorSubcoreMesh(core_axis_name="c", subcore_axis_name="s"),
    compiler_params=pltpu.CompilerParams(needs_layout_passes=False,
                                         use_tc_tiling_on_sc=True),
    scratch_types=[
        pltpu.VMEM((CTX, SUP), _i32),
        pltpu.VMEM((NSAMP, SUP), _i32),
        pltpu.VMEM((CTX * CHUNK,), _i32),
        pltpu.VMEM((NSAMP * CHUNK,), _i32),
        pltpu.VMEM((CHUNK, 128), jnp.float32),
        pltpu.VMEM((CTX * CHUNK, 128), jnp.float32),
        pltpu.VMEM((NSAMP * CHUNK, 128), jnp.float32),
        pltpu.VMEM((BW * NSAMP,), jnp.float32),
        pltpu.SemaphoreType.DMA,
    ],
)(_b_body)


def kernel(doc_ids, context_ids, sample_ids, paragraph_matrix, word_matrix,
           outputs):
    f32 = jnp.float32
    doc32 = doc_ids.astype(_i32)
    jsorted, order = lax.sort_key_val(doc32, jnp.arange(B, dtype=_i32))
    edges = jnp.concatenate([
        jnp.arange(0, 977 * 1024, 1024, dtype=_i32),
        jnp.array([999936, NDOC], dtype=_i32)])
    bnd = jnp.searchsorted(jsorted, edges).astype(_i32)
    bnd = jnp.pad(bnd, (0, BNDPAD - bnd.shape[0]), constant_values=B)
    auxp = jnp.concatenate(
        [paragraph_matrix[999936:].T, jnp.zeros((D, 64), f32)], axis=1)
    auxw = jnp.concatenate(
        [word_matrix[99968:].T, jnp.zeros((D, 96), f32)], axis=1)
    auxo = jnp.concatenate(
        [outputs[:, 99968:], jnp.zeros((D, 96), f32)], axis=1)
    doc_stage, word_rep, out_rep = _a_call(
        paragraph_matrix.T, word_matrix.T, outputs, auxp, auxw, auxo,
        jsorted, order, bnd)
    res = _b_call(context_ids.T, sample_ids.T, doc_stage, word_rep, out_rep)
    return res.reshape(B, NSAMP)


# speedup vs baseline: 2.3951x; 1.8691x over previous
import functools

import jax
import jax.numpy as jnp
from jax import lax
from jax.experimental import pallas as pl
from jax.experimental.pallas import tpu as pltpu
from jax.experimental.pallas import tpu_sc as plsc

B = 16384
CTX = 8
NSAMP = 10
D = 64
NDOC = 1000000
NWORD = 100000
NC = 2
NS = 16
NW = NC * NS
BW = B // NW

WREP = NWORD + 96
DSTG = B + NW
W_NWIN = 99
D_NWIN = 978
BNDPAD = 1088

_i32 = jnp.int32


def _splat(x):
    return jnp.full((16,), x, _i32)


def _a_body(parT, auxp, jsort, bsort, bnd, doc_stage,
            slab, asm_doc, sidx, jsv, bsv, bndv, sem):
    wid = lax.axis_index("s") * NC + lax.axis_index("c")
    lanes = lax.iota(_i32, 16)

    pltpu.sync_copy(bnd.at[pl.ds(wid * 32, 64)], bndv)
    bvecs = [bndv[pl.ds(q * 16, 16)] for q in range(3)]
    lo_t = jnp.sum(jnp.where(lanes == 0, bvecs[0], 0))
    stage0 = jnp.clip(lo_t & ~7, 0, B - 2048)
    stage0 = pl.multiple_of(stage0, 8)
    pltpu.sync_copy(jsort.at[pl.ds(stage0, 2048)], jsv)
    pltpu.sync_copy(bsort.at[pl.ds(stage0, 2048)], bsv)

    def ext(i):
        r = jnp.zeros((), _i32)
        for q in range(3):
            r = r + jnp.sum(jnp.where(lanes + 16 * q == i, bvecs[q], 0))
        return r

    def dwin_body(m, carry):
        gw = wid * 32 + m

        @pl.when(gw < D_NWIN)
        def _():
            lo = ext(m)
            hi = ext(m + 1)
            w = jnp.where(gw <= 975, gw * 1024,
                          jnp.where(gw == 976, 998912, 999936))
            w = pl.multiple_of(w, 128)

            @pl.when(gw <= 976)
            def _():
                pltpu.sync_copy(parT.at[:, pl.ds(w, 1024)],
                                slab.at[:, pl.ds(0, 1024)])

            @pl.when(gw == 977)
            def _():
                pltpu.sync_copy(auxp, slab.at[:, pl.ds(0, 128)])

            nb = (hi - lo + 15) // 16

            def batch_body(q, c2):
                pos = lo + q * 16
                off = pos - stage0
                jv = jsv[pl.ds(off, 16)]
                bv = bsv[pl.ds(off, 16)]
                msk = (pos + lanes) < hi
                jloc = jnp.clip(jv - w, 0, 1023)
                for d in range(D):
                    g = plsc.load_gather(slab, [_splat(d), jloc])
                    plsc.store_scatter(asm_doc, [lanes, _splat(d)], g)
                sidx[...] = jnp.where(msk, bv, B + wid)
                pltpu.sync_copy(asm_doc.at[:, pl.ds(0, 128)],
                                doc_stage.at[sidx])
                return c2

            lax.fori_loop(0, nb, batch_body, 0)
        return carry

    lax.fori_loop(0, 32, dwin_body, 0)


_a_call = functools.partial(
    pl.kernel,
    out_type=jax.ShapeDtypeStruct((DSTG, 128), jnp.float32),
    mesh=plsc.VectorSubcoreMesh(core_axis_name="c", subcore_axis_name="s"),
    compiler_params=pltpu.CompilerParams(needs_layout_passes=False,
                                         use_tc_tiling_on_sc=True),
    scratch_types=[
        pltpu.VMEM((D, 1032), jnp.float32),
        pltpu.VMEM((16, 136), jnp.float32),
        pltpu.VMEM((16,), _i32),
        pltpu.VMEM((2048,), _i32),
        pltpu.VMEM((2048,), _i32),
        pltpu.VMEM((64,), _i32),
        pltpu.SemaphoreType.DMA,
    ],
)(_a_body)


SUP = 128
CHUNK = 32
NSUPB = BW // SUP
KPS = SUP // CHUNK


def _b_body(ctx_hbm, smp_hbm, doc_stage, word_rep, out_rep, res_hbm,
            ctx_raw, smp_raw, ctx_idx, smp_idx,
            doc_rows, ctx_rows, smp_rows, res_v, sem):
    wid = lax.axis_index("s") * NC + lax.axis_index("c")
    lanes = lax.iota(_i32, 16)

    def sup_body(j, carry):
        base = pl.multiple_of(wid * BW + j * SUP, SUP)
        pltpu.sync_copy(ctx_hbm.at[:, pl.ds(base, SUP)], ctx_raw)
        pltpu.sync_copy(smp_hbm.at[:, pl.ds(base, SUP)], smp_raw)

        for k in range(KPS):
            for t in range(CHUNK // 16):
                for c in range(CTX):
                    ctx_idx[pl.ds(c * CHUNK + t * 16, 16)] = (
                        ctx_raw[c, pl.ds(k * CHUNK + t * 16, 16)])
                for s in range(NSAMP):
                    smp_idx[pl.ds(s * CHUNK + t * 16, 16)] = (
                        smp_raw[s, pl.ds(k * CHUNK + t * 16, 16)])
            copies = [pltpu.async_copy(
                doc_stage.at[pl.ds(base + k * CHUNK, CHUNK)], doc_rows, sem)]
            for c in range(CTX):
                copies.append(pltpu.async_copy(
                    word_rep.at[ctx_idx.at[pl.ds(c * CHUNK, CHUNK)]],
                    ctx_rows.at[pl.ds(c * CHUNK, CHUNK)], sem))
            for s in range(NSAMP):
                copies.append(pltpu.async_copy(
                    out_rep.at[smp_idx.at[pl.ds(s * CHUNK, CHUNK)]],
                    smp_rows.at[pl.ds(s * CHUNK, CHUNK)], sem))
            for cp in copies:
                cp.wait()

            rb0 = (j * SUP + k * CHUNK) * NSAMP

            def elem_body(e, c2):
                inp = []
                for t in range(4):
                    a = ctx_rows[e, pl.ds(t * 16, 16)]
                    for c in range(1, CTX):
                        a = a + ctx_rows[c * CHUNK + e, pl.ds(t * 16, 16)]
                    inp.append(doc_rows[e, pl.ds(t * 16, 16)]
                               + a * (1.0 / CTX))
                acc = jnp.zeros((16,), jnp.float32)
                for s in range(NSAMP):
                    r = inp[0] * smp_rows[s * CHUNK + e, pl.ds(0, 16)]
                    for t in range(1, 4):
                        r = r + inp[t] * smp_rows[s * CHUNK + e,
                                                  pl.ds(t * 16, 16)]
                    acc = jnp.where(lanes == s, jnp.sum(r), acc)
                plsc.store_scatter(res_v, [rb0 + e * NSAMP + lanes], acc,
                                   mask=lanes < NSAMP)
                return c2

            lax.fori_loop(0, CHUNK, elem_body, 0)
        return carry

    lax.fori_loop(0, NSUPB, sup_body, 0)
    pltpu.sync_copy(res_v,
                    res_hbm.at[pl.ds(wid * (BW * NSAMP), BW * NSAMP)])


_b_call = functools.partial(
    pl.kernel,
    out_type=jax.ShapeDtypeStruct((B * NSAMP,), jnp.float32),
    mesh=plsc.VectorSubcoreMesh(core_axis_name="c", subcore_axis_name="s"),
    compiler_params=pltpu.CompilerParams(needs_layout_passes=False,
                                         use_tc_tiling_on_sc=True),
    scratch_types=[
        pltpu.VMEM((CTX, SUP), _i32),
        pltpu.VMEM((NSAMP, SUP), _i32),
        pltpu.VMEM((CTX * CHUNK,), _i32),
        pltpu.VMEM((NSAMP * CHUNK,), _i32),
        pltpu.VMEM((CHUNK, 128), jnp.float32),
        pltpu.VMEM((CTX * CHUNK, 128), jnp.float32),
        pltpu.VMEM((NSAMP * CHUNK, 128), jnp.float32),
        pltpu.VMEM((BW * NSAMP,), jnp.float32),
        pltpu.SemaphoreType.DMA,
    ],
)(_b_body)


def kernel(doc_ids, context_ids, sample_ids, paragraph_matrix, word_matrix,
           outputs):
    f32 = jnp.float32
    doc32 = doc_ids.astype(_i32)
    jsorted, order = lax.sort_key_val(doc32, jnp.arange(B, dtype=_i32))
    edges = jnp.concatenate([
        jnp.arange(0, 977 * 1024, 1024, dtype=_i32),
        jnp.array([999936, NDOC], dtype=_i32)])
    bnd = jnp.searchsorted(jsorted, edges).astype(_i32)
    bnd = jnp.pad(bnd, (0, BNDPAD - bnd.shape[0]), constant_values=B)
    auxp = jnp.concatenate(
        [paragraph_matrix[999936:].T, jnp.zeros((D, 64), f32)], axis=1)
    word_rep = jnp.concatenate([word_matrix, word_matrix], axis=1)
    outT = outputs.T
    out_rep = jnp.concatenate([outT, outT], axis=1)
    doc_stage = _a_call(paragraph_matrix.T, auxp, jsorted, order, bnd)
    res = _b_call(context_ids.T, sample_ids.T, doc_stage, word_rep, out_rep)
    return res.reshape(B, NSAMP)


# speedup vs baseline: 2.8999x; 1.2108x over previous
import functools

import jax
import jax.numpy as jnp
from jax import lax
from jax.experimental import pallas as pl
from jax.experimental.pallas import tpu as pltpu
from jax.experimental.pallas import tpu_sc as plsc

B = 16384
CTX = 8
NSAMP = 10
D = 64
NDOC = 1000000
NWORD = 100000
NC = 2
NS = 16
NW = NC * NS
BW = B // NW

WREP = NWORD + 96
DSTG = B + NW
W_NWIN = 99
D_NWIN = 978
BNDPAD = 1088

_i32 = jnp.int32


def _splat(x):
    return jnp.full((16,), x, _i32)


def _a_body(parT, auxp, jsort, bsort, bnd, doc_stage,
            slab, asm_doc, sidx, jsv, bsv, bndv, sem):
    wid = lax.axis_index("s") * NC + lax.axis_index("c")
    lanes = lax.iota(_i32, 16)

    pltpu.sync_copy(bnd.at[pl.ds(wid * 32, 64)], bndv)
    bvecs = [bndv[pl.ds(q * 16, 16)] for q in range(3)]
    lo_t = jnp.sum(jnp.where(lanes == 0, bvecs[0], 0))
    stage0 = jnp.clip(lo_t & ~7, 0, B - 2048)
    stage0 = pl.multiple_of(stage0, 8)
    pltpu.sync_copy(jsort.at[pl.ds(stage0, 2048)], jsv)
    pltpu.sync_copy(bsort.at[pl.ds(stage0, 2048)], bsv)

    def ext(i):
        r = jnp.zeros((), _i32)
        for q in range(3):
            r = r + jnp.sum(jnp.where(lanes + 16 * q == i, bvecs[q], 0))
        return r

    def dwin_body(m, carry):
        gw = wid * 32 + m

        @pl.when(gw < D_NWIN)
        def _():
            lo = ext(m)
            hi = ext(m + 1)
            w = jnp.where(gw <= 975, gw * 1024,
                          jnp.where(gw == 976, 998912, 999936))
            w = pl.multiple_of(w, 128)

            @pl.when(gw <= 976)
            def _():
                pltpu.sync_copy(parT.at[:, pl.ds(w, 1024)],
                                slab.at[:, pl.ds(0, 1024)])

            @pl.when(gw == 977)
            def _():
                pltpu.sync_copy(auxp, slab.at[:, pl.ds(0, 128)])

            nb = (hi - lo + 15) // 16

            def batch_body(q, c2):
                pos = lo + q * 16
                off = pos - stage0
                jv = jsv[pl.ds(off, 16)]
                bv = bsv[pl.ds(off, 16)]
                msk = (pos + lanes) < hi
                jloc = jnp.clip(jv - w, 0, 1023)
                for d in range(D):
                    g = plsc.load_gather(slab, [_splat(d), jloc])
                    plsc.store_scatter(asm_doc, [lanes, _splat(d)], g)
                sidx[...] = jnp.where(msk, bv, B + wid)
                pltpu.sync_copy(asm_doc.at[:, pl.ds(0, 128)],
                                doc_stage.at[sidx])
                return c2

            lax.fori_loop(0, nb, batch_body, 0)
        return carry

    lax.fori_loop(0, 32, dwin_body, 0)


_a_call = functools.partial(
    pl.kernel,
    out_type=jax.ShapeDtypeStruct((DSTG, 128), jnp.float32),
    mesh=plsc.VectorSubcoreMesh(core_axis_name="c", subcore_axis_name="s"),
    compiler_params=pltpu.CompilerParams(needs_layout_passes=False,
                                         use_tc_tiling_on_sc=True),
    scratch_types=[
        pltpu.VMEM((D, 1032), jnp.float32),
        pltpu.VMEM((16, 136), jnp.float32),
        pltpu.VMEM((16,), _i32),
        pltpu.VMEM((2048,), _i32),
        pltpu.VMEM((2048,), _i32),
        pltpu.VMEM((64,), _i32),
        pltpu.SemaphoreType.DMA,
    ],
)(_a_body)


SUP = 128
CHUNK = 32
NSUPB = BW // SUP
KPS = SUP // CHUNK


def _b_body(ctx_hbm, smp_hbm, doc_stage, word_rep, out_rep, res_hbm,
            ctx_raw, smp_raw, ctx_idx, smp_idx,
            doc_rows, ctx_rows, smp_rows, res_v, sem):
    wid = lax.axis_index("s") * NC + lax.axis_index("c")
    lanes = lax.iota(_i32, 16)

    def sup_body(j, carry):
        base = pl.multiple_of(wid * BW + j * SUP, SUP)
        pltpu.sync_copy(ctx_hbm.at[:, pl.ds(base, SUP)], ctx_raw)
        pltpu.sync_copy(smp_hbm.at[:, pl.ds(base, SUP)], smp_raw)

        for k in range(KPS):
            for t in range(CHUNK // 16):
                for c in range(CTX):
                    ctx_idx[pl.ds(c * CHUNK + t * 16, 16)] = (
                        ctx_raw[c, pl.ds(k * CHUNK + t * 16, 16)])
                for s in range(NSAMP):
                    smp_idx[pl.ds(s * CHUNK + t * 16, 16)] = (
                        smp_raw[s, pl.ds(k * CHUNK + t * 16, 16)])
            copies = [pltpu.async_copy(
                doc_stage.at[pl.ds(base + k * CHUNK, CHUNK)], doc_rows, sem)]
            for c in range(CTX):
                copies.append(pltpu.async_copy(
                    word_rep.at[ctx_idx.at[pl.ds(c * CHUNK, CHUNK)]],
                    ctx_rows.at[pl.ds(c * CHUNK, CHUNK)], sem))
            for s in range(NSAMP):
                copies.append(pltpu.async_copy(
                    out_rep.at[smp_idx.at[pl.ds(s * CHUNK, CHUNK)]],
                    smp_rows.at[pl.ds(s * CHUNK, CHUNK)], sem))
            for cp in copies:
                cp.wait()

            rb0 = (j * SUP + k * CHUNK) * NSAMP

            def elem_body(e, c2):
                inp = []
                for t in range(4):
                    a = ctx_rows[e, pl.ds(t * 16, 16)]
                    for c in range(1, CTX):
                        a = a + ctx_rows[c * CHUNK + e, pl.ds(t * 16, 16)]
                    inp.append(doc_rows[e, pl.ds(t * 16, 16)]
                               + a * (1.0 / CTX))
                acc = jnp.zeros((16,), jnp.float32)
                for s in range(NSAMP):
                    r = inp[0] * smp_rows[s * CHUNK + e, pl.ds(0, 16)]
                    for t in range(1, 4):
                        r = r + inp[t] * smp_rows[s * CHUNK + e,
                                                  pl.ds(t * 16, 16)]
                    acc = jnp.where(lanes == s, jnp.sum(r), acc)
                plsc.store_scatter(res_v, [rb0 + e * NSAMP + lanes], acc,
                                   mask=lanes < NSAMP)
                return c2

            lax.fori_loop(0, CHUNK, elem_body, 0)
        return carry

    lax.fori_loop(0, NSUPB, sup_body, 0)
    pltpu.sync_copy(res_v,
                    res_hbm.at[pl.ds(wid * (BW * NSAMP), BW * NSAMP)])


_b_call = functools.partial(
    pl.kernel,
    out_type=jax.ShapeDtypeStruct((B * NSAMP,), jnp.float32),
    mesh=plsc.VectorSubcoreMesh(core_axis_name="c", subcore_axis_name="s"),
    compiler_params=pltpu.CompilerParams(needs_layout_passes=False,
                                         use_tc_tiling_on_sc=True),
    scratch_types=[
        pltpu.VMEM((CTX, SUP), _i32),
        pltpu.VMEM((NSAMP, SUP), _i32),
        pltpu.VMEM((CTX * CHUNK,), _i32),
        pltpu.VMEM((NSAMP * CHUNK,), _i32),
        pltpu.VMEM((CHUNK, 128), jnp.float32),
        pltpu.VMEM((CTX * CHUNK, 128), jnp.float32),
        pltpu.VMEM((NSAMP * CHUNK, 128), jnp.float32),
        pltpu.VMEM((BW * NSAMP,), jnp.float32),
        pltpu.SemaphoreType.DMA,
    ],
)(_b_body)


def kernel(doc_ids, context_ids, sample_ids, paragraph_matrix, word_matrix,
           outputs):
    f32 = jnp.float32
    doc32 = doc_ids.astype(_i32)
    jsorted, order = lax.sort_key_val(doc32, jnp.arange(B, dtype=_i32))
    edges = jnp.concatenate([
        jnp.arange(0, 977 * 1024, 1024, dtype=_i32),
        jnp.array([999936, NDOC], dtype=_i32)])
    bnd = jnp.sum(jsorted[None, :] < edges[:, None], axis=1,
                  dtype=_i32)
    bnd = jnp.pad(bnd, (0, BNDPAD - bnd.shape[0]), constant_values=B)
    auxp = jnp.concatenate(
        [paragraph_matrix[999936:].T, jnp.zeros((D, 64), f32)], axis=1)
    word_rep = jnp.concatenate([word_matrix, word_matrix], axis=1)
    outT = outputs.T
    out_rep = jnp.concatenate([outT, outT], axis=1)
    doc_stage = _a_call(paragraph_matrix.T, auxp, jsorted, order, bnd)
    res = _b_call(context_ids.T, sample_ids.T, doc_stage, word_rep, out_rep)
    return res.reshape(B, NSAMP)
